# baseline (speedup 1.0000x reference)
import jax
import jax.numpy as jnp
from jax import lax
from jax.experimental import pallas as pl
from jax.experimental.pallas import tpu as pltpu
from jax.experimental.pallas import tpu_sc as plsc

NUM_HASHES = 2
EMBED_DIM = 64
VOCAB = 1000000
RATIO = 4

NC = 2
NS = 16
NW = NC * NS
LANES = 16

CHUNK = 64
IDXS = CHUNK * NUM_HASHES


def _sc_body(x_hbm, wt_hbm, emb_hbm, out_hbm,
             xv, idx0, idx1, widx0, widx1,
             rows0, rows1, wv0, wv1, outb0, outb1,
             sem_r0, sem_r1, sem_w0, sem_w1, sem_o0, sem_o1):
    n_pairs = out_hbm.shape[0]
    np_w = n_pairs // NW
    n_sub = np_w // CHUNK

    wid = lax.axis_index("s") * NC + lax.axis_index("c")
    pair0 = wid * np_w

    idxb = (idx0, idx1)
    widxb = (widx0, widx1)
    rows = (rows0, rows1)
    wv = (wv0, wv1)
    outb = (outb0, outb1)
    sem_r = (sem_r0, sem_r1)
    sem_w = (sem_w0, sem_w1)
    sem_o = (sem_o0, sem_o1)

    iota = lax.iota(jnp.int32, LANES)
    offv = (iota & 1) * jnp.int32(VOCAB + 1)

    pltpu.sync_copy(x_hbm.at[pl.ds(pair0 * NUM_HASHES, np_w * NUM_HASHES)], xv)

    def prep_and_fire(s, b):
        for i in range(IDXS // LANES):
            xi = xv[pl.ds(s * IDXS + i * LANES, LANES)]
            idxb[b][pl.ds(i * LANES, LANES)] = xi >> 2
            widxb[b][pl.ds(i * LANES, LANES)] = xi + offv
        pltpu.async_copy(emb_hbm.at[idxb[b]], rows[b], sem_r[b])
        pltpu.async_copy(wt_hbm.at[widxb[b]], wv[b], sem_w[b])

    prep_and_fire(0, 0)
    prep_and_fire(1, 1)

    def compute(b):
        for g in range(CHUNK // LANES):
            pj = g * LANES + iota
            rj = pj * 2
            w0 = plsc.load_gather(wv[b], [rj])
            w1 = plsc.load_gather(wv[b], [rj + 1])
            for d in range(EMBED_DIM):
                dv = jnp.full((LANES,), d, jnp.int32)
                e0 = plsc.load_gather(rows[b], [rj, dv])
                e1 = plsc.load_gather(rows[b], [rj + 1, dv])
                plsc.store_scatter(outb[b], [pj, dv], w0 * e0 + w1 * e1)

    def loop_body(t, carry):
        for b in range(2):
            s = t * 2 + b
            pltpu.make_async_copy(emb_hbm.at[idxb[b]], rows[b], sem_r[b]).wait()
            pltpu.make_async_copy(wt_hbm.at[widxb[b]], wv[b], sem_w[b]).wait()
            dst = out_hbm.at[pl.ds(pair0 + s * CHUNK, CHUNK)]

            @pl.when(s >= 2)
            def _():
                pltpu.make_async_copy(outb[b], dst, sem_o[b]).wait()

            compute(b)
            pltpu.async_copy(outb[b], dst, sem_o[b])

            @pl.when(s + 2 < n_sub)
            def _():
                prep_and_fire(s + 2, b)
        return carry

    lax.fori_loop(0, n_sub // 2, loop_body, 0)

    for b in range(2):
        s_last = n_sub - 2 + b
        dst = out_hbm.at[pl.ds(pair0 + s_last * CHUNK, CHUNK)]
        pltpu.make_async_copy(outb[b], dst, sem_o[b]).wait()


def kernel(x, weight_table, emb_table):
    b_sz, seq, h = x.shape
    n_pairs = b_sz * seq
    xf = x.reshape(n_pairs * h)
    wt = weight_table.reshape(-1)

    mesh = plsc.VectorSubcoreMesh(core_axis_name="c", subcore_axis_name="s",
                                  num_cores=NC, num_subcores=NS)
    np_w = n_pairs // NW

    f = pl.kernel(
        _sc_body,
        out_type=jax.ShapeDtypeStruct((n_pairs, EMBED_DIM), jnp.float32),
        mesh=mesh,
        compiler_params=pltpu.CompilerParams(needs_layout_passes=False,
                                             use_tc_tiling_on_sc=False),
        scratch_types=[
            pltpu.VMEM((np_w * NUM_HASHES,), jnp.int32),
            pltpu.VMEM((IDXS,), jnp.int32),
            pltpu.VMEM((IDXS,), jnp.int32),
            pltpu.VMEM((IDXS,), jnp.int32),
            pltpu.VMEM((IDXS,), jnp.int32),
            pltpu.VMEM((IDXS, EMBED_DIM), jnp.float32),
            pltpu.VMEM((IDXS, EMBED_DIM), jnp.float32),
            pltpu.VMEM((IDXS,), jnp.float32),
            pltpu.VMEM((IDXS,), jnp.float32),
            pltpu.VMEM((CHUNK, EMBED_DIM), jnp.float32),
            pltpu.VMEM((CHUNK, EMBED_DIM), jnp.float32),
            pltpu.SemaphoreType.DMA,
            pltpu.SemaphoreType.DMA,
            pltpu.SemaphoreType.DMA,
            pltpu.SemaphoreType.DMA,
            pltpu.SemaphoreType.DMA,
            pltpu.SemaphoreType.DMA,
        ],
    )
    out = f(xf, wt, emb_table)
    return out.reshape(b_sz, seq, EMBED_DIM)
